# baseline (speedup 1.0000x reference)
import functools

import jax
import jax.numpy as jnp
from jax import lax
from jax.experimental import pallas as pl
from jax.experimental.pallas import tpu as pltpu
from jax.experimental.pallas import tpu_sc as plsc

_NUM_WORKERS = 32
_LANES = 16


def _paged_update(tok_k, tok_v, slot_mapping, k_cache, v_cache):
    n_tok = tok_k.shape[0]
    num_slots = k_cache.shape[0]
    chunk = num_slots // _NUM_WORKERS
    mesh = plsc.VectorSubcoreMesh(core_axis_name="c", subcore_axis_name="s")

    @functools.partial(
        pl.kernel,
        mesh=mesh,
        out_type=(
            jax.ShapeDtypeStruct(k_cache.shape, k_cache.dtype),
            jax.ShapeDtypeStruct(v_cache.shape, v_cache.dtype),
        ),
        scratch_types=[
            pltpu.VMEM((n_tok,), jnp.int32),
            pltpu.SemaphoreType.DMA,
            pltpu.SemaphoreType.DMA,
        ],
    )
    def body(tok_k_hbm, tok_v_hbm, slot_hbm, kc_in, vc_in, kc_out, vc_out,
             idx_v, sem_k, sem_v):
        wid = lax.axis_index("s") * 2 + lax.axis_index("c")
        base = wid * chunk
        rng = pl.ds(base, chunk)
        copy_k = pltpu.async_copy(kc_in.at[rng], kc_out.at[rng], sem_k)
        copy_v = pltpu.async_copy(vc_in.at[rng], vc_out.at[rng], sem_v)
        pltpu.sync_copy(slot_hbm, idx_v)
        copy_k.wait()
        copy_v.wait()
        for c in range(n_tok // _LANES):
            vec = idx_v[pl.ds(c * _LANES, _LANES)]
            for lane in range(_LANES):
                i = c * _LANES + lane
                s = vec[lane]

                @pl.when((s >= base) & (s < base + chunk))
                def _():
                    pltpu.sync_copy(tok_k_hbm.at[i], kc_out.at[s])
                    pltpu.sync_copy(tok_v_hbm.at[i], vc_out.at[s])

    return body(tok_k, tok_v, slot_mapping, k_cache, v_cache)


def kernel(pos_ids, k_val, v_val, slot_mapping, batch_idx, k_cache, v_cache):
    B, H, S, D = k_val.shape
    tok_k = jnp.transpose(k_val, (0, 2, 1, 3)).reshape(B * S, H, D)
    tok_v = jnp.transpose(v_val, (0, 2, 1, 3)).reshape(B * S, H, D)
    return _paged_update(tok_k, tok_v, slot_mapping, k_cache, v_cache)


# speedup vs baseline: 16.1151x; 16.1151x over previous
import functools

import jax
import jax.numpy as jnp
from jax import lax
from jax.experimental import pallas as pl
from jax.experimental.pallas import tpu as pltpu

_BLOCK = 1024


def _copy_scatter(slots, tok_k, tok_v, k_cache, v_cache):
    n_tok = tok_k.shape[0]
    num_slots = k_cache.shape[0]
    n_heads, head_dim = k_cache.shape[1], k_cache.shape[2]
    grid = (num_slots // _BLOCK,)

    def body(slots_ref, kc_ref, vc_ref, tk_ref, tv_ref, ko_ref, vo_ref):
        i = pl.program_id(0)
        base = i * _BLOCK
        ko_ref[...] = kc_ref[...]
        vo_ref[...] = vc_ref[...]

        def tok(t, carry):
            s = slots_ref[t]

            @pl.when((s >= base) & (s < base + _BLOCK))
            def _():
                r = s - base
                ko_ref[pl.ds(r, 1)] = tk_ref[pl.ds(t, 1)]
                vo_ref[pl.ds(r, 1)] = tv_ref[pl.ds(t, 1)]

            return carry

        lax.fori_loop(0, n_tok, tok, 0)

    blk = pl.BlockSpec((_BLOCK, n_heads, head_dim), lambda i, s: (i, 0, 0))
    tokblk = pl.BlockSpec((n_tok, n_heads, head_dim), lambda i, s: (0, 0, 0))
    return pl.pallas_call(
        body,
        grid_spec=pltpu.PrefetchScalarGridSpec(
            num_scalar_prefetch=1,
            grid=grid,
            in_specs=[blk, blk, tokblk, tokblk],
            out_specs=[blk, blk],
        ),
        out_shape=(
            jax.ShapeDtypeStruct(k_cache.shape, k_cache.dtype),
            jax.ShapeDtypeStruct(v_cache.shape, v_cache.dtype),
        ),
        compiler_params=pltpu.CompilerParams(
            dimension_semantics=("arbitrary",),
        ),
    )(slots, k_cache, v_cache, tok_k, tok_v)


def kernel(pos_ids, k_val, v_val, slot_mapping, batch_idx, k_cache, v_cache):
    B, H, S, D = k_val.shape
    tok_k = jnp.transpose(k_val, (0, 2, 1, 3)).reshape(B * S, H, D)
    tok_v = jnp.transpose(v_val, (0, 2, 1, 3)).reshape(B * S, H, D)
    return _copy_scatter(slot_mapping, tok_k, tok_v, k_cache, v_cache)
